# Initial kernel scaffold; baseline (speedup 1.0000x reference)
import jax
import jax.numpy as jnp
from jax.experimental import pallas as pl


def kernel(x):
    raise NotImplementedError("write your pallas kernel here")


# speedup vs baseline: 1.2093x; 1.2093x over previous
import numpy as np
import jax
import jax.numpy as jnp
from jax.experimental import pallas as pl

_ANCHOR_W = (116.0, 156.0, 373.0)
_ANCHOR_H = (90.0, 198.0, 326.0)
_NG = 64
_NCELL = _NG * _NG
_ATTRS_IN = 87
_ATTRS_OUT = 86
_STRIDE = 512.0 / _NG


def _decode_body(x_ref, o_ref):
    a = pl.program_id(1)
    t = x_ref[0].reshape(_ATTRS_IN, _NCELL)

    xs = jax.nn.sigmoid(t[0:1])
    ys = jax.nn.sigmoid(t[1:2])
    ew = jnp.exp(t[2:3])
    el = jnp.exp(t[3:4])
    sin2 = jnp.tanh(t[4:5])
    cos2 = jnp.tanh(t[5:6])
    conf_cls = jax.nn.sigmoid(t[6:_ATTRS_IN])

    cell = jax.lax.broadcasted_iota(jnp.int32, (1, _NCELL), 1)
    gx = (cell % _NG).astype(jnp.float32)
    gy = (cell // _NG).astype(jnp.float32)

    af = a.astype(jnp.float32)
    aw = jnp.where(a == 0, _ANCHOR_W[0] / _STRIDE,
                   jnp.where(a == 1, _ANCHOR_W[1] / _STRIDE,
                             _ANCHOR_W[2] / _STRIDE))
    ah = jnp.where(a == 0, _ANCHOR_H[0] / _STRIDE,
                   jnp.where(a == 1, _ANCHOR_H[1] / _STRIDE,
                             _ANCHOR_H[2] / _STRIDE))
    del af

    px = (xs + gx) * _STRIDE
    py = (ys + gy) * _STRIDE
    pw = ew * (aw * _STRIDE)
    plh = el * (ah * _STRIDE)
    theta = jnp.arctan2(sin2, cos2) * (90.0 / np.pi)

    pad = jnp.zeros((2, _NCELL), dtype=jnp.float32)
    slab = jnp.concatenate([px, py, pw, plh, theta, conf_cls, pad], axis=0)
    o_ref[0] = slab.T[:, :_ATTRS_OUT]


def kernel(x):
    nB = x.shape[0]
    out_shape = jax.ShapeDtypeStruct((nB, 3 * _NCELL, _ATTRS_OUT), jnp.float32)
    return pl.pallas_call(
        _decode_body,
        grid=(nB, 3),
        in_specs=[
            pl.BlockSpec((1, _ATTRS_IN, _NG, _NG), lambda b, a: (b, a, 0, 0)),
        ],
        out_specs=pl.BlockSpec((1, _NCELL, _ATTRS_OUT), lambda b, a: (b, a, 0)),
        out_shape=out_shape,
    )(x)
